# Initial kernel scaffold; baseline (speedup 1.0000x reference)
import jax
import jax.numpy as jnp
from jax.experimental import pallas as pl


def kernel(xyz1, xyz2, points1, points2, W1, b1, W2, b2):
    raise NotImplementedError("write your pallas kernel here")


# speedup vs baseline: 23.5208x; 23.5208x over previous
import functools

import jax
import jax.numpy as jnp
from jax import lax
from jax.experimental import pallas as pl

BLK = 512


def _fp_kernel(x1_ref, x2t_ref, p1_ref, p2_ref, w1_ref, b1_ref, w2_ref,
               b2_ref, out_ref, *, n2, c2):
    x1 = x1_ref[0]
    x2t = x2t_ref[0]

    sq1 = (x1[:, 0:1] * x1[:, 0:1] + x1[:, 1:2] * x1[:, 1:2]
           + x1[:, 2:3] * x1[:, 2:3])
    sq2 = (x2t[0:1] * x2t[0:1] + x2t[1:2] * x2t[1:2]
           + x2t[2:3] * x2t[2:3])
    dot = jnp.dot(x1.astype(jnp.bfloat16), x2t.astype(jnp.bfloat16),
                  preferred_element_type=jnp.float32)
    d = sq1 + sq2 - 2.0 * dot

    iota = lax.broadcasted_iota(jnp.int32, d.shape, 1)
    wsp = jnp.zeros_like(d)
    norm = jnp.zeros((d.shape[0], 1), dtype=jnp.float32)
    for _ in range(3):
        v = jnp.min(d, axis=1, keepdims=True)
        idx = jnp.argmin(d, axis=1)[:, None]
        onehot = (iota == idx).astype(jnp.float32)
        r = 1.0 / jnp.maximum(v, 1e-10)
        wsp = wsp + r * onehot
        norm = norm + r
        d = jnp.where(iota == idx, jnp.inf, d)

    wsp = wsp / norm

    interp = jnp.dot(wsp, p2_ref[0], preferred_element_type=jnp.float32)

    w1 = w1_ref[...]
    h = (jnp.dot(interp, w1[:c2], preferred_element_type=jnp.float32)
         + jnp.dot(p1_ref[0], w1[c2:], preferred_element_type=jnp.float32)
         + b1_ref[...])
    h = jnp.maximum(h, 0.0)
    out = jnp.dot(h, w2_ref[...], preferred_element_type=jnp.float32)
    out = jnp.maximum(out + b2_ref[...], 0.0)
    out_ref[0] = out


def kernel(xyz1, xyz2, points1, points2, W1, b1, W2, b2):
    B, N1, _ = xyz1.shape
    N2 = xyz2.shape[1]
    C1 = points1.shape[2]
    C2 = points2.shape[2]
    CH = W1.shape[1]
    CO = W2.shape[1]

    xyz2t = jnp.transpose(xyz2, (0, 2, 1))
    b1r = b1.reshape(1, CH)
    b2r = b2.reshape(1, CO)

    grid = (B, N1 // BLK)
    out = pl.pallas_call(
        functools.partial(_fp_kernel, n2=N2, c2=C2),
        grid=grid,
        in_specs=[
            pl.BlockSpec((1, BLK, 3), lambda b, n: (b, n, 0)),
            pl.BlockSpec((1, 3, N2), lambda b, n: (b, 0, 0)),
            pl.BlockSpec((1, BLK, C1), lambda b, n: (b, n, 0)),
            pl.BlockSpec((1, N2, C2), lambda b, n: (b, 0, 0)),
            pl.BlockSpec((C1 + C2, CH), lambda b, n: (0, 0)),
            pl.BlockSpec((1, CH), lambda b, n: (0, 0)),
            pl.BlockSpec((CH, CO), lambda b, n: (0, 0)),
            pl.BlockSpec((1, CO), lambda b, n: (0, 0)),
        ],
        out_specs=pl.BlockSpec((1, BLK, CO), lambda b, n: (b, n, 0)),
        out_shape=jax.ShapeDtypeStruct((B, N1, CO), jnp.float32),
    )(xyz1, xyz2t, points1, points2, W1, b1r, W2, b2r)
    return out


# speedup vs baseline: 44.9738x; 1.9121x over previous
import functools

import jax
import jax.numpy as jnp
from jax import lax
from jax.experimental import pallas as pl

BLK = 512


def _fp_kernel(x1_ref, x2t_ref, p1_ref, p2_ref, w1_ref, b1_ref, w2_ref,
               b2_ref, out_ref, *, n2, c2):
    x1 = x1_ref[0]
    x2t = x2t_ref[0]

    sq1 = (x1[:, 0:1] * x1[:, 0:1] + x1[:, 1:2] * x1[:, 1:2]
           + x1[:, 2:3] * x1[:, 2:3])
    sq2 = (x2t[0:1] * x2t[0:1] + x2t[1:2] * x2t[1:2]
           + x2t[2:3] * x2t[2:3])
    dot = jnp.dot(x1.astype(jnp.bfloat16), x2t.astype(jnp.bfloat16),
                  preferred_element_type=jnp.float32)
    d = sq1 + sq2 - 2.0 * dot

    v1 = jnp.min(d, axis=1, keepdims=True)
    d2 = jnp.where(d > v1, d, jnp.inf)
    v2 = jnp.min(d2, axis=1, keepdims=True)
    d3 = jnp.where(d2 > v2, d2, jnp.inf)
    v3 = jnp.min(d3, axis=1, keepdims=True)
    recip = 1.0 / jnp.maximum(d, 1e-10)
    wsp = jnp.where(d <= v3, recip, 0.0)
    norm = jnp.sum(wsp, axis=1, keepdims=True)

    interp = jnp.dot(wsp, p2_ref[0], preferred_element_type=jnp.float32)
    interp = interp / norm

    w1 = w1_ref[...]
    h = (jnp.dot(interp, w1[:c2], preferred_element_type=jnp.float32)
         + jnp.dot(p1_ref[0], w1[c2:], preferred_element_type=jnp.float32)
         + b1_ref[...])
    h = jnp.maximum(h, 0.0)
    out = jnp.dot(h, w2_ref[...], preferred_element_type=jnp.float32)
    out = jnp.maximum(out + b2_ref[...], 0.0)
    out_ref[0] = out


def kernel(xyz1, xyz2, points1, points2, W1, b1, W2, b2):
    B, N1, _ = xyz1.shape
    N2 = xyz2.shape[1]
    C1 = points1.shape[2]
    C2 = points2.shape[2]
    CH = W1.shape[1]
    CO = W2.shape[1]

    xyz2t = jnp.transpose(xyz2, (0, 2, 1))
    b1r = b1.reshape(1, CH)
    b2r = b2.reshape(1, CO)

    grid = (B, N1 // BLK)
    out = pl.pallas_call(
        functools.partial(_fp_kernel, n2=N2, c2=C2),
        grid=grid,
        in_specs=[
            pl.BlockSpec((1, BLK, 3), lambda b, n: (b, n, 0)),
            pl.BlockSpec((1, 3, N2), lambda b, n: (b, 0, 0)),
            pl.BlockSpec((1, BLK, C1), lambda b, n: (b, n, 0)),
            pl.BlockSpec((1, N2, C2), lambda b, n: (b, 0, 0)),
            pl.BlockSpec((C1 + C2, CH), lambda b, n: (0, 0)),
            pl.BlockSpec((1, CH), lambda b, n: (0, 0)),
            pl.BlockSpec((CH, CO), lambda b, n: (0, 0)),
            pl.BlockSpec((1, CO), lambda b, n: (0, 0)),
        ],
        out_specs=pl.BlockSpec((1, BLK, CO), lambda b, n: (b, n, 0)),
        out_shape=jax.ShapeDtypeStruct((B, N1, CO), jnp.float32),
    )(xyz1, xyz2t, points1, points2, W1, b1r, W2, b2r)
    return out


# speedup vs baseline: 48.9853x; 1.0892x over previous
import functools

import jax
import jax.numpy as jnp
from jax import lax
from jax.experimental import pallas as pl

BLK = 512


def _fp_kernel(x1_ref, x2t_ref, p1_ref, p2_ref, w1_ref, b1_ref, w2_ref,
               b2_ref, out_ref, *, n2, c2):
    x1 = x1_ref[0]
    x2t = x2t_ref[0]

    sq1 = (x1[:, 0:1] * x1[:, 0:1] + x1[:, 1:2] * x1[:, 1:2]
           + x1[:, 2:3] * x1[:, 2:3])
    sq2 = (x2t[0:1] * x2t[0:1] + x2t[1:2] * x2t[1:2]
           + x2t[2:3] * x2t[2:3])
    dot2 = jnp.dot((-2.0 * x1).astype(jnp.bfloat16), x2t.astype(jnp.bfloat16),
                   preferred_element_type=jnp.float32)
    d = (sq1 + sq2) + dot2

    v1 = jnp.min(d, axis=1, keepdims=True)
    d2 = jnp.where(d > v1, d, jnp.inf)
    v2 = jnp.min(d2, axis=1, keepdims=True)
    d3 = jnp.where(d2 > v2, d2, jnp.inf)
    v3 = jnp.min(d3, axis=1, keepdims=True)
    recip = 1.0 / jnp.maximum(d, 1e-10)
    wsp = jnp.where(d <= v3, recip, 0.0)
    norm = (1.0 / jnp.maximum(v1, 1e-10) + 1.0 / jnp.maximum(v2, 1e-10)
            + 1.0 / jnp.maximum(v3, 1e-10))

    interp = jnp.dot(wsp, p2_ref[0], preferred_element_type=jnp.float32)
    interp = interp / norm

    w1 = w1_ref[...]
    h = (jnp.dot(interp.astype(jnp.bfloat16), w1[:c2].astype(jnp.bfloat16),
                 preferred_element_type=jnp.float32)
         + jnp.dot(p1_ref[0].astype(jnp.bfloat16),
                   w1[c2:].astype(jnp.bfloat16),
                   preferred_element_type=jnp.float32)
         + b1_ref[...])
    h = jnp.maximum(h, 0.0)
    out = jnp.dot(h.astype(jnp.bfloat16), w2_ref[...].astype(jnp.bfloat16),
                  preferred_element_type=jnp.float32)
    out = jnp.maximum(out + b2_ref[...], 0.0)
    out_ref[0] = out


def kernel(xyz1, xyz2, points1, points2, W1, b1, W2, b2):
    B, N1, _ = xyz1.shape
    N2 = xyz2.shape[1]
    C1 = points1.shape[2]
    C2 = points2.shape[2]
    CH = W1.shape[1]
    CO = W2.shape[1]

    xyz2t = jnp.transpose(xyz2, (0, 2, 1))
    b1r = b1.reshape(1, CH)
    b2r = b2.reshape(1, CO)

    grid = (B, N1 // BLK)
    out = pl.pallas_call(
        functools.partial(_fp_kernel, n2=N2, c2=C2),
        grid=grid,
        in_specs=[
            pl.BlockSpec((1, BLK, 3), lambda b, n: (b, n, 0)),
            pl.BlockSpec((1, 3, N2), lambda b, n: (b, 0, 0)),
            pl.BlockSpec((1, BLK, C1), lambda b, n: (b, n, 0)),
            pl.BlockSpec((1, N2, C2), lambda b, n: (b, 0, 0)),
            pl.BlockSpec((C1 + C2, CH), lambda b, n: (0, 0)),
            pl.BlockSpec((1, CH), lambda b, n: (0, 0)),
            pl.BlockSpec((CH, CO), lambda b, n: (0, 0)),
            pl.BlockSpec((1, CO), lambda b, n: (0, 0)),
        ],
        out_specs=pl.BlockSpec((1, BLK, CO), lambda b, n: (b, n, 0)),
        out_shape=jax.ShapeDtypeStruct((B, N1, CO), jnp.float32),
    )(xyz1, xyz2t, points1, points2, W1, b1r, W2, b2r)
    return out
